# Initial kernel scaffold; baseline (speedup 1.0000x reference)
import jax
import jax.numpy as jnp
from jax.experimental import pallas as pl


def kernel(input, weight):
    raise NotImplementedError("write your pallas kernel here")


# speedup vs baseline: 1.6846x; 1.6846x over previous
import functools

import jax
import jax.numpy as jnp
from jax import lax
from jax.experimental import pallas as pl
from jax.experimental.pallas import tpu as pltpu
from jax.experimental.pallas import tpu_sc as plsc

_info = plsc.get_sparse_core_info()
_NC, _NS = _info.num_cores, _info.num_subcores
_NW = _NC * _NS

_CHUNK = 128


def _make_gather(vocab, dim, batch):
  assert batch % (_NW * _CHUNK) == 0
  b_per_w = batch // _NW
  n_chunks = b_per_w // _CHUNK
  mesh = plsc.VectorSubcoreMesh(core_axis_name="c", subcore_axis_name="s")

  @functools.partial(
      pl.kernel,
      mesh=mesh,
      out_type=jax.ShapeDtypeStruct((batch, dim), jnp.float32),
      scratch_types=[
          pltpu.VMEM((n_chunks, _CHUNK), jnp.int32),
          pltpu.VMEM((_CHUNK, dim), jnp.float32),
          pltpu.SemaphoreType.DMA,
      ],
      compiler_params=pltpu.CompilerParams(use_tc_tiling_on_sc=False),
  )
  def gather_kernel(idx_hbm, table_hbm, out_hbm, idx_v, rows_v, gsem):
    wid = lax.axis_index("s") * _NC + lax.axis_index("c")
    base = wid * b_per_w
    pltpu.sync_copy(idx_hbm.at[wid], idx_v)

    def step(j, carry):
      pltpu.async_copy(table_hbm.at[idx_v.at[j]], rows_v, gsem).wait()
      pltpu.sync_copy(rows_v, out_hbm.at[pl.ds(base + j * _CHUNK, _CHUNK)])
      return carry

    lax.fori_loop(0, n_chunks, step, 0)

  return gather_kernel


def kernel(input, weight):
  batch, hist = input.shape
  vocab, dim = weight.shape
  flat = batch * hist
  idx = input.reshape(_NW, flat // (_NW * _CHUNK), _CHUNK).astype(jnp.int32)
  out = _make_gather(vocab, dim, flat)(idx, weight)
  return out.reshape(batch, hist, dim)


# speedup vs baseline: 1.8771x; 1.1142x over previous
import functools

import jax
import jax.numpy as jnp
from jax import lax
from jax.experimental import pallas as pl
from jax.experimental.pallas import tpu as pltpu
from jax.experimental.pallas import tpu_sc as plsc

_info = plsc.get_sparse_core_info()
_NC, _NS = _info.num_cores, _info.num_subcores
_NW = _NC * _NS

_CHUNK = 128


_NBUF = 4


def _make_gather(vocab, dim, batch):
  assert batch % (_NW * _CHUNK) == 0
  b_per_w = batch // _NW
  n_chunks = b_per_w // _CHUNK
  assert n_chunks % _NBUF == 0
  n_outer = n_chunks // _NBUF
  mesh = plsc.VectorSubcoreMesh(core_axis_name="c", subcore_axis_name="s")

  @functools.partial(
      pl.kernel,
      mesh=mesh,
      out_type=jax.ShapeDtypeStruct((batch, dim), jnp.float32),
      scratch_types=[
          pltpu.VMEM((n_chunks, _CHUNK), jnp.int32),
          pltpu.VMEM((_NBUF, _CHUNK, dim), jnp.float32),
      ]
      + [pltpu.SemaphoreType.DMA] * (2 * _NBUF),
      compiler_params=pltpu.CompilerParams(use_tc_tiling_on_sc=False),
  )
  def gather_kernel(idx_hbm, table_hbm, out_hbm, idx_v, rows_v, *sems):
    gsem = sems[:_NBUF]
    ssem = sems[_NBUF:]
    wid = lax.axis_index("s") * _NC + lax.axis_index("c")
    base = wid * b_per_w
    pltpu.sync_copy(idx_hbm.at[wid], idx_v)

    for b in range(_NBUF):
      pltpu.async_copy(table_hbm.at[idx_v.at[b]], rows_v.at[b], gsem[b])

    def outer(g, carry):
      for b in range(_NBUF):
        j = g * _NBUF + b
        pltpu.make_async_copy(
            table_hbm.at[idx_v.at[j]], rows_v.at[b], gsem[b]
        ).wait()
        pltpu.async_copy(
            rows_v.at[b], out_hbm.at[pl.ds(base + j * _CHUNK, _CHUNK)], ssem[b]
        )
        pltpu.make_async_copy(
            rows_v.at[b], out_hbm.at[pl.ds(base + j * _CHUNK, _CHUNK)], ssem[b]
        ).wait()

        @pl.when(g < n_outer - 1)
        def _():
          pltpu.async_copy(
              table_hbm.at[idx_v.at[j + _NBUF]], rows_v.at[b], gsem[b]
          )

      return carry

    lax.fori_loop(0, n_outer, outer, 0)

  return gather_kernel


def kernel(input, weight):
  batch, hist = input.shape
  vocab, dim = weight.shape
  flat = batch * hist
  idx = input.reshape(_NW, flat // (_NW * _CHUNK), _CHUNK).astype(jnp.int32)
  out = _make_gather(vocab, dim, flat)(idx, weight)
  return out.reshape(batch, hist, dim)
